# baseline (speedup 1.0000x reference)
import functools

import jax
import jax.numpy as jnp
from jax import lax
from jax.experimental import pallas as pl
from jax.experimental.pallas import tpu as pltpu
from jax.experimental.pallas import tpu_sc as plsc

N = 10000
E = 160000
D_IN = 256
D_H = 512

NC = 2
NS = 16
EPT = E // NS
EPC = E // (NC * NS)
WB = 128
SLAB = 10112
GB = 32
NRING = 3
EPAD = 10176
NGB = EPAD // GB
NRND = NGB // NRING


def _sc_mesh():
    return plsc.VectorSubcoreMesh(
        core_axis_name="c", subcore_axis_name="s", num_cores=NC, num_subcores=NS
    )


def _zero_zbuf(zbuf):
    zero16 = jnp.zeros((16,), jnp.float32)
    for r in range(16):
        for j in range(WB // 16):
            zbuf[r, pl.ds(j * 16, 16)] = zero16


def _zero_slab(s, zbuf, slab):
    rows = SLAB // NS
    for k in range(rows // 16):
        pltpu.sync_copy(zbuf, slab.at[pl.ds(s * rows + k * 16, 16)])
    pltpu.sync_copy(zbuf.at[pl.ds(0, 8)],
                    slab.at[pl.ds(s * rows + (rows // 16) * 16, 8)])


def _copy_out(c, s, slab, out_hbm, col):
    rows = SLAB // NS
    last = N - (NS - 1) * rows

    @pl.when(s < NS - 1)
    def _():
        pltpu.sync_copy(slab.at[pl.ds(s * rows, rows)],
                        out_hbm.at[pl.ds(c * N + s * rows, rows),
                                   pl.ds(col, WB)])

    @pl.when(s == NS - 1)
    def _():
        pltpu.sync_copy(slab.at[pl.ds((NS - 1) * rows, last)],
                        out_hbm.at[pl.ds(c * N + (NS - 1) * rows, last),
                                   pl.ds(col, WB)])


def _pad_and_adjust(srcb, dstb, wb, rowbase):
    zi = jnp.zeros((16,), jnp.int32)
    dumpv = jnp.full((16,), N, jnp.int32)
    zf = jnp.zeros((16,), jnp.float32)
    for j in range((EPAD - EPT) // 16):
        srcb[pl.ds(EPT + j * 16, 16)] = zi
        dstb[pl.ds(EPT + j * 16, 16)] = dumpv
        wb[pl.ds(EPT + j * 16, 16)] = zf

    def adj(i, carry):
        sl = pl.ds(i * 16, 16)
        srcb[sl] = srcb[sl] + rowbase
        return carry

    lax.fori_loop(0, EPAD // 16, adj, jnp.int32(0))


def _agg_pass(x_hbm, srcb, dstb, wb, gbufs, sbufs, slab, sems, ssems):
    def start_gather(bb, k):
        pltpu.async_copy(x_hbm.at[srcb.at[pl.ds(bb, GB)]], gbufs[k], sems[k])

    def wait_scatter(sl):
        pltpu.make_async_copy(x_hbm.at[pl.ds(0, 16)], sbufs[sl],
                              ssems[sl]).wait()

    for k in range(NRING):
        start_gather(k * GB, k)

    def round_body(i, carry):
        for k in range(NRING):
            bb = (i * NRING + k) * GB
            pltpu.make_async_copy(x_hbm.at[pl.ds(0, GB)], gbufs[k],
                                  sems[k]).wait()
            g = gbufs[k]
            for half in range(2):
                sl = half
                if k == 0:
                    @pl.when(i > 0)
                    def _():
                        wait_scatter(sl)
                else:
                    wait_scatter(sl)
                hb = bb + half * 16
                dv = dstb[pl.ds(hb, 16)]
                wv = wb[pl.ds(hb, 16)]
                sb = sbufs[sl]
                for r in range(16):
                    w_r = wv[r]
                    gr = half * 16 + r
                    for j in range(WB // 16):
                        sb[r, pl.ds(j * 16, 16)] = (
                            g[gr, pl.ds(j * 16, 16)] * w_r)
                pltpu.async_copy(sb, slab.at[dv], ssems[sl], add=True)

            @pl.when(i < NRND - 1)
            def _():
                start_gather(bb + NRING * GB, k)
        return carry

    lax.fori_loop(0, NRND, round_body, jnp.int32(0))
    wait_scatter(0)
    wait_scatter(1)


def _sc_deg_body(ed_hbm, out_hbm, dstb, sbuf, zbuf, slab):
    c = lax.axis_index("c")
    s = lax.axis_index("s")
    base_e = (c * NS + s) * EPC
    pltpu.sync_copy(ed_hbm.at[pl.ds(base_e, EPC)], dstb.at[pl.ds(0, EPC)])
    dstb[pl.ds(EPC, 16)] = jnp.full((16,), N, jnp.int32)

    onehot = jnp.where(lax.iota(jnp.int32, 16) == 0, 1.0, 0.0).astype(jnp.float32)
    _zero_zbuf(zbuf)
    zero16 = jnp.zeros((16,), jnp.float32)
    for r in range(16):
        sbuf[r, pl.ds(0, 16)] = onehot
        for j in range(1, WB // 16):
            sbuf[r, pl.ds(j * 16, 16)] = zero16

    _zero_slab(s, zbuf, slab)
    plsc.subcore_barrier()

    def body(b, carry):
        dv = dstb[pl.ds(b * 16, 16)]
        pltpu.sync_copy(sbuf, slab.at[dv], add=True)
        return carry

    lax.fori_loop(0, (EPC + 15) // 16, body, jnp.int32(0))
    plsc.subcore_barrier()
    _copy_out(c, s, slab, out_hbm, 0)


def _sc_deg(edst):
    f = pl.kernel(
        _sc_deg_body,
        out_type=jax.ShapeDtypeStruct((2 * N, WB), jnp.float32),
        mesh=_sc_mesh(),
        scratch_types=[
            pltpu.VMEM((EPC + 32,), jnp.int32),
            pltpu.VMEM((16, WB), jnp.float32),
            pltpu.VMEM((16, WB), jnp.float32),
            pltpu.VMEM_SHARED((SLAB, WB), jnp.float32),
        ],
    )
    return f(edst)


def _sc_agg0_body(x_hbm, es_hbm, ed_hbm, ew_hbm, out_hbm,
                  srcb, dstb, wb, g0, g1, g2, sb0, sb1, zbuf, slab,
                  s0, s1, s2, ss0, ss1):
    c = lax.axis_index("c")
    s = lax.axis_index("s")
    base_e = s * EPT
    pltpu.sync_copy(es_hbm.at[pl.ds(base_e, EPT)], srcb.at[pl.ds(0, EPT)])
    pltpu.sync_copy(ed_hbm.at[pl.ds(base_e, EPT)], dstb.at[pl.ds(0, EPT)])
    pltpu.sync_copy(ew_hbm.at[pl.ds(base_e, EPT)], wb.at[pl.ds(0, EPT)])

    _pad_and_adjust(srcb, dstb, wb, c * N)
    _zero_zbuf(zbuf)
    _zero_slab(s, zbuf, slab)
    plsc.subcore_barrier()

    _agg_pass(x_hbm, srcb, dstb, wb, [g0, g1, g2], [sb0, sb1], slab,
              [s0, s1, s2], [ss0, ss1])
    plsc.subcore_barrier()
    _copy_out(c, s, slab, out_hbm, 0)


def _sc_agg0(x2, esrc, edst, edge_w):
    f = pl.kernel(
        _sc_agg0_body,
        out_type=jax.ShapeDtypeStruct((2 * N, WB), jnp.float32),
        mesh=_sc_mesh(),
        scratch_types=(
            [pltpu.VMEM((EPAD,), jnp.int32),
             pltpu.VMEM((EPAD,), jnp.int32),
             pltpu.VMEM((EPAD,), jnp.float32)]
            + [pltpu.VMEM((GB, WB), jnp.float32)] * NRING
            + [pltpu.VMEM((16, WB), jnp.float32)] * 3
            + [pltpu.VMEM_SHARED((SLAB, WB), jnp.float32)]
            + [pltpu.SemaphoreType.DMA] * (NRING + 2)
        ),
    )
    return f(x2, esrc, edst, edge_w)


def _sc_agg_body(xa_hbm, xb_hbm, es_hbm, ed_hbm, ew_hbm, out_hbm,
                 srcb, dstb, wb, g0, g1, g2, sb0, sb1, zbuf, slab,
                 s0, s1, s2, ss0, ss1):
    c = lax.axis_index("c")
    s = lax.axis_index("s")
    base_e = s * EPT
    pltpu.sync_copy(es_hbm.at[pl.ds(base_e, EPT)], srcb.at[pl.ds(0, EPT)])
    pltpu.sync_copy(ed_hbm.at[pl.ds(base_e, EPT)], dstb.at[pl.ds(0, EPT)])
    pltpu.sync_copy(ew_hbm.at[pl.ds(base_e, EPT)], wb.at[pl.ds(0, EPT)])
    _zero_zbuf(zbuf)

    _pad_and_adjust(srcb, dstb, wb, c * N)
    for p, x_hbm in enumerate((xa_hbm, xb_hbm)):
        _zero_slab(s, zbuf, slab)
        plsc.subcore_barrier()
        _agg_pass(x_hbm, srcb, dstb, wb, [g0, g1, g2], [sb0, sb1],
                  slab, [s0, s1, s2], [ss0, ss1])
        plsc.subcore_barrier()
        _copy_out(c, s, slab, out_hbm, p * WB)
        plsc.subcore_barrier()


def _sc_agg(xa, xb, esrc, edst, edge_w):
    f = pl.kernel(
        _sc_agg_body,
        out_type=jax.ShapeDtypeStruct((2 * N, 2 * WB), jnp.float32),
        mesh=_sc_mesh(),
        scratch_types=(
            [pltpu.VMEM((EPAD,), jnp.int32),
             pltpu.VMEM((EPAD,), jnp.int32),
             pltpu.VMEM((EPAD,), jnp.float32)]
            + [pltpu.VMEM((GB, WB), jnp.float32)] * NRING
            + [pltpu.VMEM((16, WB), jnp.float32)] * 3
            + [pltpu.VMEM_SHARED((SLAB, WB), jnp.float32)]
            + [pltpu.SemaphoreType.DMA] * (NRING + 2)
        ),
    )
    return f(xa, xb, esrc, edst, edge_w)


RB = 1000
GR = N // RB


def _tc0_body(aggA, aggB, degA, degB, xA, xB, wref, bref, eref, outa, outb):
    deg = degA[0][:, :1] + degB[0][:, :1]
    rdeg = 1.0 / jnp.maximum(deg, 1.0)
    mean = jnp.concatenate([aggA[0], aggB[0]], axis=1) * rdeg
    x = jnp.concatenate([xA[0], xB[0]], axis=1)
    h = (1.0 + eref[0]) * x + mean
    y = jnp.dot(h, wref[...], preferred_element_type=jnp.float32) + bref[...]
    y = jnp.maximum(y, 0.0)
    outa[...] = y[:, :WB][None]
    outb[...] = y[:, WB:][None]


def _tc0(agg0, degp, x0, W1, b1, eps):
    return pl.pallas_call(
        _tc0_body,
        grid=(GR, 2),
        in_specs=[
            pl.BlockSpec((1, RB, WB), lambda i, c: (0, i, 0)),
            pl.BlockSpec((1, RB, WB), lambda i, c: (1, i, 0)),
            pl.BlockSpec((1, RB, WB), lambda i, c: (0, i, 0)),
            pl.BlockSpec((1, RB, WB), lambda i, c: (1, i, 0)),
            pl.BlockSpec((1, RB, WB), lambda i, c: (0, i, 0)),
            pl.BlockSpec((1, RB, WB), lambda i, c: (1, i, 0)),
            pl.BlockSpec((D_IN, 256), lambda i, c: (0, c)),
            pl.BlockSpec((1, 256), lambda i, c: (0, c)),
            pl.BlockSpec(memory_space=pltpu.SMEM),
        ],
        out_specs=[
            pl.BlockSpec((1, RB, WB), lambda i, c: (c, i, 0)),
            pl.BlockSpec((1, RB, WB), lambda i, c: (c, i, 0)),
        ],
        out_shape=[
            jax.ShapeDtypeStruct((2, N, WB), jnp.float32),
            jax.ShapeDtypeStruct((2, N, WB), jnp.float32),
        ],
    )(agg0, agg0, degp, degp, x0, x0, W1, b1, eps)


def _tc12_body(relu, zero0, li,
               aggA, aggB, degA, degB, xa0, xb0, xa1, xb1,
               wref, bref, eref, *outs):
    deg = degA[0][:, :1] + degB[0][:, :1]
    rdeg = 1.0 / jnp.maximum(deg, 1.0)
    mean = jnp.concatenate([aggA[0], aggB[0]], axis=1) * rdeg
    x = jnp.concatenate([xa0[0], xb0[0], xa1[0], xb1[0]], axis=1)
    h = (1.0 + eref[li]) * x + mean
    y = jnp.dot(h, wref[...], preferred_element_type=jnp.float32) + bref[...]
    if relu:
        y = jnp.maximum(y, 0.0)
    if zero0:
        i = pl.program_id(0)
        ri = lax.broadcasted_iota(jnp.int32, y.shape, 0)
        y = jnp.where((i == 0) & (ri == 0), 0.0, y)
        outs[0][...] = y
    else:
        outs[0][...] = y[:, :WB][None]
        outs[1][...] = y[:, WB:][None]


def _tc12(agg, xa, xb, degp, W2, b2, eps, li, relu, zero0):
    body = functools.partial(_tc12_body, relu, zero0, li)
    if zero0:
        out_specs = pl.BlockSpec((RB, 256), lambda i, c: (i, c))
        out_shape = jax.ShapeDtypeStruct((N, D_H), jnp.float32)
    else:
        out_specs = [
            pl.BlockSpec((1, RB, WB), lambda i, c: (c, i, 0)),
            pl.BlockSpec((1, RB, WB), lambda i, c: (c, i, 0)),
        ]
        out_shape = [
            jax.ShapeDtypeStruct((2, N, WB), jnp.float32),
            jax.ShapeDtypeStruct((2, N, WB), jnp.float32),
        ]
    return pl.pallas_call(
        body,
        grid=(GR, 2),
        in_specs=[
            pl.BlockSpec((1, RB, 2 * WB), lambda i, c: (0, i, 0)),
            pl.BlockSpec((1, RB, 2 * WB), lambda i, c: (1, i, 0)),
            pl.BlockSpec((1, RB, WB), lambda i, c: (0, i, 0)),
            pl.BlockSpec((1, RB, WB), lambda i, c: (1, i, 0)),
            pl.BlockSpec((1, RB, WB), lambda i, c: (0, i, 0)),
            pl.BlockSpec((1, RB, WB), lambda i, c: (0, i, 0)),
            pl.BlockSpec((1, RB, WB), lambda i, c: (1, i, 0)),
            pl.BlockSpec((1, RB, WB), lambda i, c: (1, i, 0)),
            pl.BlockSpec((D_H, 256), lambda i, c: (0, c)),
            pl.BlockSpec((1, 256), lambda i, c: (0, c)),
            pl.BlockSpec(memory_space=pltpu.SMEM),
        ],
        out_specs=out_specs,
        out_shape=out_shape,
    )(agg, agg, degp, degp, xa, xb, xa, xb, W2, b2, eps)


def kernel(ndata, edge_index, edge_w, W1, b1, W2, b2, eps):
    x0 = jnp.concatenate([ndata[:, :WB], ndata[:, WB:]], axis=0)
    b1r = b1.reshape(1, D_H)
    b2r = b2.reshape(1, D_H)

    esrc = edge_index[0]
    edst = edge_index[1]
    degp = _sc_deg(edst).reshape(2, N, WB)
    agg0 = _sc_agg0(x0, esrc, edst, edge_w)
    agg0r = agg0.reshape(2, N, WB)
    x0r = x0.reshape(2, N, WB)
    xa1, xb1 = _tc0(agg0r, degp, x0r, W1, b1r, eps)

    agg1 = _sc_agg(xa1.reshape(2 * N, WB), xb1.reshape(2 * N, WB),
                   esrc, edst, edge_w)
    xa2, xb2 = _tc12(agg1.reshape(2, N, 2 * WB), xa1, xb1, degp,
                     W2, b2r, eps, 1, True, False)

    agg2 = _sc_agg(xa2.reshape(2 * N, WB), xb2.reshape(2 * N, WB),
                   esrc, edst, edge_w)
    y = _tc12(agg2.reshape(2, N, 2 * WB), xa2, xb2, degp,
              W2, b2r, eps, 2, False, True)
    return y


# speedup vs baseline: 1.6354x; 1.6354x over previous
import functools

import jax
import jax.numpy as jnp
from jax import lax
from jax.experimental import pallas as pl
from jax.experimental.pallas import tpu as pltpu
from jax.experimental.pallas import tpu_sc as plsc

N = 10000
E = 160000
D_IN = 256
D_H = 512

NC = 2
NS = 16
EPT = E // NS
EPC = E // (NC * NS)
WB = 128
SLAB = 10112
NBUF = 5
NBAT = EPT // 16


def _sc_mesh():
    return plsc.VectorSubcoreMesh(
        core_axis_name="c", subcore_axis_name="s", num_cores=NC, num_subcores=NS
    )


def _zero_zbuf(zbuf):
    zero16 = jnp.zeros((16,), jnp.float32)
    for r in range(16):
        for j in range(WB // 16):
            zbuf[r, pl.ds(j * 16, 16)] = zero16


def _zero_slab(s, zbuf, slab):
    rows = SLAB // NS
    for k in range(rows // 16):
        pltpu.sync_copy(zbuf, slab.at[pl.ds(s * rows + k * 16, 16)])
    pltpu.sync_copy(zbuf.at[pl.ds(0, 8)],
                    slab.at[pl.ds(s * rows + (rows // 16) * 16, 8)])


def _copy_out(c, s, slab, out_hbm, col):
    rows = SLAB // NS
    last = N - (NS - 1) * rows

    @pl.when(s < NS - 1)
    def _():
        pltpu.sync_copy(slab.at[pl.ds(s * rows, rows)],
                        out_hbm.at[pl.ds(c * N + s * rows, rows),
                                   pl.ds(col, WB)])

    @pl.when(s == NS - 1)
    def _():
        pltpu.sync_copy(slab.at[pl.ds((NS - 1) * rows, last)],
                        out_hbm.at[pl.ds(c * N + (NS - 1) * rows, last),
                                   pl.ds(col, WB)])


def _agg_pass(x_hbm, srcb, dstb, wb, gbufs, sbufs, slab, rowbase, sems, ssems):
    def start_gather(bb, k):
        sv = srcb[pl.ds(bb, 16)] + rowbase
        pltpu.async_copy(x_hbm.at[sv], gbufs[k], sems[k])

    def wait_scatter(sl):
        pltpu.make_async_copy(x_hbm.at[pl.ds(0, 16)], sbufs[sl],
                              ssems[sl]).wait()

    for k in range(NBUF):
        start_gather(k * 16, k)

    def round_body(i, carry):
        for k in range(NBUF):
            sl = k % 2
            bb = (i * NBUF + k) * 16
            pltpu.make_async_copy(x_hbm.at[pl.ds(0, 16)], gbufs[k],
                                  sems[k]).wait()
            if k < 2:
                @pl.when(i > 0)
                def _():
                    wait_scatter(sl)
            else:
                wait_scatter(sl)
            dv = dstb[pl.ds(bb, 16)]
            wv = wb[pl.ds(bb, 16)]
            g = gbufs[k]
            sb = sbufs[sl]
            for r in range(16):
                w_r = wv[r]
                for j in range(WB // 16):
                    sb[r, pl.ds(j * 16, 16)] = g[r, pl.ds(j * 16, 16)] * w_r
            pltpu.async_copy(sb, slab.at[dv], ssems[sl], add=True)

            @pl.when(i < NBAT // NBUF - 1)
            def _():
                start_gather(bb + NBUF * 16, k)
        return carry

    lax.fori_loop(0, NBAT // NBUF, round_body, jnp.int32(0))
    wait_scatter(0)
    wait_scatter(1)


def _sc_deg_body(ed_hbm, out_hbm, dstb, sbuf, zbuf, slab):
    c = lax.axis_index("c")
    s = lax.axis_index("s")
    base_e = (c * NS + s) * EPC
    pltpu.sync_copy(ed_hbm.at[pl.ds(base_e, EPC)], dstb.at[pl.ds(0, EPC)])
    dstb[pl.ds(EPC, 16)] = jnp.full((16,), N, jnp.int32)

    onehot = jnp.where(lax.iota(jnp.int32, 16) == 0, 1.0, 0.0).astype(jnp.float32)
    _zero_zbuf(zbuf)
    zero16 = jnp.zeros((16,), jnp.float32)
    for r in range(16):
        sbuf[r, pl.ds(0, 16)] = onehot
        for j in range(1, WB // 16):
            sbuf[r, pl.ds(j * 16, 16)] = zero16

    _zero_slab(s, zbuf, slab)
    plsc.subcore_barrier()

    def body(b, carry):
        dv = dstb[pl.ds(b * 16, 16)]
        pltpu.sync_copy(sbuf, slab.at[dv], add=True)
        return carry

    lax.fori_loop(0, (EPC + 15) // 16, body, jnp.int32(0))
    plsc.subcore_barrier()
    _copy_out(c, s, slab, out_hbm, 0)


def _sc_deg(edst):
    f = pl.kernel(
        _sc_deg_body,
        out_type=jax.ShapeDtypeStruct((2 * N, WB), jnp.float32),
        mesh=_sc_mesh(),
        scratch_types=[
            pltpu.VMEM((EPC + 32,), jnp.int32),
            pltpu.VMEM((16, WB), jnp.float32),
            pltpu.VMEM((16, WB), jnp.float32),
            pltpu.VMEM_SHARED((SLAB, WB), jnp.float32),
        ],
    )
    return f(edst)


def _sc_agg0_body(x_hbm, es_hbm, ed_hbm, ew_hbm, out_hbm,
                  srcb, dstb, wb, g0, g1, g2, g3, g4, sb0, sb1, zbuf, slab,
                  s0, s1, s2, s3, s4, ss0, ss1):
    c = lax.axis_index("c")
    s = lax.axis_index("s")
    base_e = s * EPT
    pltpu.sync_copy(es_hbm.at[pl.ds(base_e, EPT)], srcb)
    pltpu.sync_copy(ed_hbm.at[pl.ds(base_e, EPT)], dstb)
    pltpu.sync_copy(ew_hbm.at[pl.ds(base_e, EPT)], wb)

    _zero_zbuf(zbuf)
    _zero_slab(s, zbuf, slab)
    plsc.subcore_barrier()

    _agg_pass(x_hbm, srcb, dstb, wb, [g0, g1, g2, g3, g4], [sb0, sb1], slab,
              c * N, [s0, s1, s2, s3, s4], [ss0, ss1])
    plsc.subcore_barrier()
    _copy_out(c, s, slab, out_hbm, 0)


def _sc_agg0(x2, esrc, edst, edge_w):
    f = pl.kernel(
        _sc_agg0_body,
        out_type=jax.ShapeDtypeStruct((2 * N, WB), jnp.float32),
        mesh=_sc_mesh(),
        scratch_types=(
            [pltpu.VMEM((EPT,), jnp.int32),
             pltpu.VMEM((EPT,), jnp.int32),
             pltpu.VMEM((EPT,), jnp.float32)]
            + [pltpu.VMEM((16, WB), jnp.float32)] * (NBUF + 3)
            + [pltpu.VMEM_SHARED((SLAB, WB), jnp.float32)]
            + [pltpu.SemaphoreType.DMA] * (NBUF + 2)
        ),
    )
    return f(x2, esrc, edst, edge_w)


def _sc_agg_body(xa_hbm, xb_hbm, es_hbm, ed_hbm, ew_hbm, out_hbm,
                 srcb, dstb, wb, g0, g1, g2, g3, g4, sb0, sb1, zbuf, slab,
                 s0, s1, s2, s3, s4, ss0, ss1):
    c = lax.axis_index("c")
    s = lax.axis_index("s")
    base_e = s * EPT
    pltpu.sync_copy(es_hbm.at[pl.ds(base_e, EPT)], srcb)
    pltpu.sync_copy(ed_hbm.at[pl.ds(base_e, EPT)], dstb)
    pltpu.sync_copy(ew_hbm.at[pl.ds(base_e, EPT)], wb)
    _zero_zbuf(zbuf)

    for p, x_hbm in enumerate((xa_hbm, xb_hbm)):
        _zero_slab(s, zbuf, slab)
        plsc.subcore_barrier()
        _agg_pass(x_hbm, srcb, dstb, wb, [g0, g1, g2, g3, g4], [sb0, sb1],
                  slab, c * N, [s0, s1, s2, s3, s4], [ss0, ss1])
        plsc.subcore_barrier()
        _copy_out(c, s, slab, out_hbm, p * WB)
        plsc.subcore_barrier()


def _sc_agg(xa, xb, esrc, edst, edge_w):
    f = pl.kernel(
        _sc_agg_body,
        out_type=jax.ShapeDtypeStruct((2 * N, 2 * WB), jnp.float32),
        mesh=_sc_mesh(),
        scratch_types=(
            [pltpu.VMEM((EPT,), jnp.int32),
             pltpu.VMEM((EPT,), jnp.int32),
             pltpu.VMEM((EPT,), jnp.float32)]
            + [pltpu.VMEM((16, WB), jnp.float32)] * (NBUF + 3)
            + [pltpu.VMEM_SHARED((SLAB, WB), jnp.float32)]
            + [pltpu.SemaphoreType.DMA] * (NBUF + 2)
        ),
    )
    return f(xa, xb, esrc, edst, edge_w)


RB = 1000
GR = N // RB


def _tc0_body(aggA, aggB, degA, degB, xA, xB, wref, bref, eref, outa, outb):
    deg = degA[0][:, :1] + degB[0][:, :1]
    rdeg = 1.0 / jnp.maximum(deg, 1.0)
    mean = jnp.concatenate([aggA[0], aggB[0]], axis=1) * rdeg
    x = jnp.concatenate([xA[0], xB[0]], axis=1)
    h = (1.0 + eref[0]) * x + mean
    y = jnp.dot(h, wref[...], preferred_element_type=jnp.float32) + bref[...]
    y = jnp.maximum(y, 0.0)
    outa[...] = y[:, :WB][None]
    outb[...] = y[:, WB:][None]


def _tc0(agg0, degp, x0, W1, b1, eps):
    return pl.pallas_call(
        _tc0_body,
        grid=(GR, 2),
        in_specs=[
            pl.BlockSpec((1, RB, WB), lambda i, c: (0, i, 0)),
            pl.BlockSpec((1, RB, WB), lambda i, c: (1, i, 0)),
            pl.BlockSpec((1, RB, WB), lambda i, c: (0, i, 0)),
            pl.BlockSpec((1, RB, WB), lambda i, c: (1, i, 0)),
            pl.BlockSpec((1, RB, WB), lambda i, c: (0, i, 0)),
            pl.BlockSpec((1, RB, WB), lambda i, c: (1, i, 0)),
            pl.BlockSpec((D_IN, 256), lambda i, c: (0, c)),
            pl.BlockSpec((1, 256), lambda i, c: (0, c)),
            pl.BlockSpec(memory_space=pltpu.SMEM),
        ],
        out_specs=[
            pl.BlockSpec((1, RB, WB), lambda i, c: (c, i, 0)),
            pl.BlockSpec((1, RB, WB), lambda i, c: (c, i, 0)),
        ],
        out_shape=[
            jax.ShapeDtypeStruct((2, N, WB), jnp.float32),
            jax.ShapeDtypeStruct((2, N, WB), jnp.float32),
        ],
    )(agg0, agg0, degp, degp, x0, x0, W1, b1, eps)


def _tc12_body(relu, zero0, li,
               aggA, aggB, degA, degB, xa0, xb0, xa1, xb1,
               wref, bref, eref, *outs):
    deg = degA[0][:, :1] + degB[0][:, :1]
    rdeg = 1.0 / jnp.maximum(deg, 1.0)
    mean = jnp.concatenate([aggA[0], aggB[0]], axis=1) * rdeg
    x = jnp.concatenate([xa0[0], xb0[0], xa1[0], xb1[0]], axis=1)
    h = (1.0 + eref[li]) * x + mean
    y = jnp.dot(h, wref[...], preferred_element_type=jnp.float32) + bref[...]
    if relu:
        y = jnp.maximum(y, 0.0)
    if zero0:
        i = pl.program_id(0)
        ri = lax.broadcasted_iota(jnp.int32, y.shape, 0)
        y = jnp.where((i == 0) & (ri == 0), 0.0, y)
        outs[0][...] = y
    else:
        outs[0][...] = y[:, :WB][None]
        outs[1][...] = y[:, WB:][None]


def _tc12(agg, xa, xb, degp, W2, b2, eps, li, relu, zero0):
    body = functools.partial(_tc12_body, relu, zero0, li)
    if zero0:
        out_specs = pl.BlockSpec((RB, 256), lambda i, c: (i, c))
        out_shape = jax.ShapeDtypeStruct((N, D_H), jnp.float32)
    else:
        out_specs = [
            pl.BlockSpec((1, RB, WB), lambda i, c: (c, i, 0)),
            pl.BlockSpec((1, RB, WB), lambda i, c: (c, i, 0)),
        ]
        out_shape = [
            jax.ShapeDtypeStruct((2, N, WB), jnp.float32),
            jax.ShapeDtypeStruct((2, N, WB), jnp.float32),
        ]
    return pl.pallas_call(
        body,
        grid=(GR, 2),
        in_specs=[
            pl.BlockSpec((1, RB, 2 * WB), lambda i, c: (0, i, 0)),
            pl.BlockSpec((1, RB, 2 * WB), lambda i, c: (1, i, 0)),
            pl.BlockSpec((1, RB, WB), lambda i, c: (0, i, 0)),
            pl.BlockSpec((1, RB, WB), lambda i, c: (1, i, 0)),
            pl.BlockSpec((1, RB, WB), lambda i, c: (0, i, 0)),
            pl.BlockSpec((1, RB, WB), lambda i, c: (0, i, 0)),
            pl.BlockSpec((1, RB, WB), lambda i, c: (1, i, 0)),
            pl.BlockSpec((1, RB, WB), lambda i, c: (1, i, 0)),
            pl.BlockSpec((D_H, 256), lambda i, c: (0, c)),
            pl.BlockSpec((1, 256), lambda i, c: (0, c)),
            pl.BlockSpec(memory_space=pltpu.SMEM),
        ],
        out_specs=out_specs,
        out_shape=out_shape,
    )(agg, agg, degp, degp, xa, xb, xa, xb, W2, b2, eps)


def kernel(ndata, edge_index, edge_w, W1, b1, W2, b2, eps):
    x0 = jnp.concatenate([ndata[:, :WB], ndata[:, WB:]], axis=0)
    b1r = b1.reshape(1, D_H)
    b2r = b2.reshape(1, D_H)

    esrc = edge_index[0]
    edst = edge_index[1]
    degp = _sc_deg(edst).reshape(2, N, WB)
    agg0 = _sc_agg0(x0, esrc, edst, edge_w)
    agg0r = agg0.reshape(2, N, WB)
    x0r = x0.reshape(2, N, WB)
    xa1, xb1 = _tc0(agg0r, degp, x0r, W1, b1r, eps)

    agg1 = _sc_agg(xa1.reshape(2 * N, WB), xb1.reshape(2 * N, WB),
                   esrc, edst, edge_w)
    xa2, xb2 = _tc12(agg1.reshape(2, N, 2 * WB), xa1, xb1, degp,
                     W2, b2r, eps, 1, True, False)

    agg2 = _sc_agg(xa2.reshape(2 * N, WB), xb2.reshape(2 * N, WB),
                   esrc, edst, edge_w)
    y = _tc12(agg2.reshape(2, N, 2 * WB), xa2, xb2, degp,
              W2, b2r, eps, 2, False, True)
    return y


# speedup vs baseline: 1.6466x; 1.0068x over previous
import functools

import jax
import jax.numpy as jnp
from jax import lax
from jax.experimental import pallas as pl
from jax.experimental.pallas import tpu as pltpu
from jax.experimental.pallas import tpu_sc as plsc

N = 10000
E = 160000
D_IN = 256
D_H = 512

NC = 2
NS = 16
EPT = E // NS
EPC = E // (NC * NS)
WB = 128
SLAB = 10112
NBUF = 5
NBAT = EPT // 16


def _sc_mesh():
    return plsc.VectorSubcoreMesh(
        core_axis_name="c", subcore_axis_name="s", num_cores=NC, num_subcores=NS
    )


def _zero_zbuf(zbuf):
    zero16 = jnp.zeros((16,), jnp.float32)
    for r in range(16):
        for j in range(WB // 16):
            zbuf[r, pl.ds(j * 16, 16)] = zero16


def _zero_slab(s, zbuf, slab):
    rows = SLAB // NS
    for k in range(rows // 16):
        pltpu.sync_copy(zbuf, slab.at[pl.ds(s * rows + k * 16, 16)])
    pltpu.sync_copy(zbuf.at[pl.ds(0, 8)],
                    slab.at[pl.ds(s * rows + (rows // 16) * 16, 8)])


def _copy_out(c, s, slab, out_hbm, col):
    rows = SLAB // NS
    last = N - (NS - 1) * rows

    @pl.when(s < NS - 1)
    def _():
        pltpu.sync_copy(slab.at[pl.ds(s * rows, rows)],
                        out_hbm.at[pl.ds(c * N + s * rows, rows),
                                   pl.ds(col, WB)])

    @pl.when(s == NS - 1)
    def _():
        pltpu.sync_copy(slab.at[pl.ds((NS - 1) * rows, last)],
                        out_hbm.at[pl.ds(c * N + (NS - 1) * rows, last),
                                   pl.ds(col, WB)])


def _agg_pass(x_hbm, srcb, dstb, wb, gbufs, sbufs, slab, rowbase, sems, ssems):
    def start_gather(bb, k):
        sv = srcb[pl.ds(bb, 16)] + rowbase
        pltpu.async_copy(x_hbm.at[sv], gbufs[k], sems[k])

    def wait_scatter(sl):
        pltpu.make_async_copy(x_hbm.at[pl.ds(0, 16)], sbufs[sl],
                              ssems[sl]).wait()

    for k in range(NBUF):
        start_gather(k * 16, k)

    def round_body(i, carry):
        for k in range(NBUF):
            sl = k % 2
            bb = (i * NBUF + k) * 16
            pltpu.make_async_copy(x_hbm.at[pl.ds(0, 16)], gbufs[k],
                                  sems[k]).wait()
            if k < 2:
                @pl.when(i > 0)
                def _():
                    wait_scatter(sl)
            else:
                wait_scatter(sl)
            dv = dstb[pl.ds(bb, 16)]
            wv = wb[pl.ds(bb, 16)]
            g = gbufs[k]
            sb = sbufs[sl]
            for r in range(16):
                w_r = wv[r]
                for j in range(WB // 16):
                    sb[r, pl.ds(j * 16, 16)] = g[r, pl.ds(j * 16, 16)] * w_r
            pltpu.async_copy(sb, slab.at[dv], ssems[sl], add=True)

            @pl.when(i < NBAT // NBUF - 1)
            def _():
                start_gather(bb + NBUF * 16, k)
        return carry

    lax.fori_loop(0, NBAT // NBUF, round_body, jnp.int32(0))
    wait_scatter(0)
    wait_scatter(1)


def _sc_agg0_body(x_hbm, es_hbm, ed_hbm, ew_hbm, deg_hbm, out_hbm,
                  srcb, dstb, wb, g0, g1, g2, g3, g4, sb0, sb1, zbuf, slab,
                  s0, s1, s2, s3, s4, ss0, ss1):
    c = lax.axis_index("c")
    s = lax.axis_index("s")

    base_d = (c * NS + s) * EPC
    pltpu.sync_copy(ed_hbm.at[pl.ds(base_d, EPC)], dstb.at[pl.ds(0, EPC)])
    dstb[pl.ds(EPC, 16)] = jnp.full((16,), N, jnp.int32)
    onehot = jnp.where(lax.iota(jnp.int32, 16) == 0, 1.0, 0.0).astype(jnp.float32)
    zero16 = jnp.zeros((16,), jnp.float32)
    for r in range(16):
        sb0[r, pl.ds(0, 16)] = onehot
        for j in range(1, WB // 16):
            sb0[r, pl.ds(j * 16, 16)] = zero16
    _zero_zbuf(zbuf)
    _zero_slab(s, zbuf, slab)
    plsc.subcore_barrier()

    def dbody(b, carry):
        dv = dstb[pl.ds(b * 16, 16)]
        pltpu.sync_copy(sb0, slab.at[dv], add=True)
        return carry

    lax.fori_loop(0, (EPC + 15) // 16, dbody, jnp.int32(0))
    plsc.subcore_barrier()
    _copy_out(c, s, slab, deg_hbm, 0)
    plsc.subcore_barrier()

    base_e = s * EPT
    pltpu.sync_copy(es_hbm.at[pl.ds(base_e, EPT)], srcb)
    pltpu.sync_copy(ed_hbm.at[pl.ds(base_e, EPT)], dstb)
    pltpu.sync_copy(ew_hbm.at[pl.ds(base_e, EPT)], wb)

    _zero_slab(s, zbuf, slab)
    plsc.subcore_barrier()

    _agg_pass(x_hbm, srcb, dstb, wb, [g0, g1, g2, g3, g4], [sb0, sb1], slab,
              c * N, [s0, s1, s2, s3, s4], [ss0, ss1])
    plsc.subcore_barrier()
    _copy_out(c, s, slab, out_hbm, 0)


def _sc_agg0(x2, esrc, edst, edge_w):
    f = pl.kernel(
        _sc_agg0_body,
        out_type=[jax.ShapeDtypeStruct((2 * N, WB), jnp.float32),
                  jax.ShapeDtypeStruct((2 * N, WB), jnp.float32)],
        mesh=_sc_mesh(),
        scratch_types=(
            [pltpu.VMEM((EPT,), jnp.int32),
             pltpu.VMEM((EPT,), jnp.int32),
             pltpu.VMEM((EPT,), jnp.float32)]
            + [pltpu.VMEM((16, WB), jnp.float32)] * (NBUF + 3)
            + [pltpu.VMEM_SHARED((SLAB, WB), jnp.float32)]
            + [pltpu.SemaphoreType.DMA] * (NBUF + 2)
        ),
    )
    return f(x2, esrc, edst, edge_w)


def _sc_agg_body(xa_hbm, xb_hbm, es_hbm, ed_hbm, ew_hbm, out_hbm,
                 srcb, dstb, wb, g0, g1, g2, g3, g4, sb0, sb1, zbuf, slab,
                 s0, s1, s2, s3, s4, ss0, ss1):
    c = lax.axis_index("c")
    s = lax.axis_index("s")
    base_e = s * EPT
    pltpu.sync_copy(es_hbm.at[pl.ds(base_e, EPT)], srcb)
    pltpu.sync_copy(ed_hbm.at[pl.ds(base_e, EPT)], dstb)
    pltpu.sync_copy(ew_hbm.at[pl.ds(base_e, EPT)], wb)
    _zero_zbuf(zbuf)

    for p, x_hbm in enumerate((xa_hbm, xb_hbm)):
        _zero_slab(s, zbuf, slab)
        plsc.subcore_barrier()
        _agg_pass(x_hbm, srcb, dstb, wb, [g0, g1, g2, g3, g4], [sb0, sb1],
                  slab, c * N, [s0, s1, s2, s3, s4], [ss0, ss1])
        plsc.subcore_barrier()
        _copy_out(c, s, slab, out_hbm, p * WB)
        plsc.subcore_barrier()


def _sc_agg(xa, xb, esrc, edst, edge_w):
    f = pl.kernel(
        _sc_agg_body,
        out_type=jax.ShapeDtypeStruct((2 * N, 2 * WB), jnp.float32),
        mesh=_sc_mesh(),
        scratch_types=(
            [pltpu.VMEM((EPT,), jnp.int32),
             pltpu.VMEM((EPT,), jnp.int32),
             pltpu.VMEM((EPT,), jnp.float32)]
            + [pltpu.VMEM((16, WB), jnp.float32)] * (NBUF + 3)
            + [pltpu.VMEM_SHARED((SLAB, WB), jnp.float32)]
            + [pltpu.SemaphoreType.DMA] * (NBUF + 2)
        ),
    )
    return f(xa, xb, esrc, edst, edge_w)


RB = 1000
GR = N // RB


def _tc0_body(aggA, aggB, degA, degB, xA, xB, wref, bref, eref, outa, outb):
    deg = degA[0][:, :1] + degB[0][:, :1]
    rdeg = 1.0 / jnp.maximum(deg, 1.0)
    mean = jnp.concatenate([aggA[0], aggB[0]], axis=1) * rdeg
    x = jnp.concatenate([xA[0], xB[0]], axis=1)
    h = (1.0 + eref[0]) * x + mean
    y = jnp.dot(h, wref[...], preferred_element_type=jnp.float32) + bref[...]
    y = jnp.maximum(y, 0.0)
    outa[...] = y[:, :WB][None]
    outb[...] = y[:, WB:][None]


def _tc0(agg0, degp, x0, W1, b1, eps):
    return pl.pallas_call(
        _tc0_body,
        grid=(GR, 2),
        in_specs=[
            pl.BlockSpec((1, RB, WB), lambda i, c: (0, i, 0)),
            pl.BlockSpec((1, RB, WB), lambda i, c: (1, i, 0)),
            pl.BlockSpec((1, RB, WB), lambda i, c: (0, i, 0)),
            pl.BlockSpec((1, RB, WB), lambda i, c: (1, i, 0)),
            pl.BlockSpec((1, RB, WB), lambda i, c: (0, i, 0)),
            pl.BlockSpec((1, RB, WB), lambda i, c: (1, i, 0)),
            pl.BlockSpec((D_IN, 256), lambda i, c: (0, c)),
            pl.BlockSpec((1, 256), lambda i, c: (0, c)),
            pl.BlockSpec(memory_space=pltpu.SMEM),
        ],
        out_specs=[
            pl.BlockSpec((1, RB, WB), lambda i, c: (c, i, 0)),
            pl.BlockSpec((1, RB, WB), lambda i, c: (c, i, 0)),
        ],
        out_shape=[
            jax.ShapeDtypeStruct((2, N, WB), jnp.float32),
            jax.ShapeDtypeStruct((2, N, WB), jnp.float32),
        ],
    )(agg0, agg0, degp, degp, x0, x0, W1, b1, eps)


def _tc12_body(relu, zero0, li,
               aggA, aggB, degA, degB, xa0, xb0, xa1, xb1,
               wref, bref, eref, *outs):
    deg = degA[0][:, :1] + degB[0][:, :1]
    rdeg = 1.0 / jnp.maximum(deg, 1.0)
    mean = jnp.concatenate([aggA[0], aggB[0]], axis=1) * rdeg
    x = jnp.concatenate([xa0[0], xb0[0], xa1[0], xb1[0]], axis=1)
    h = (1.0 + eref[li]) * x + mean
    y = jnp.dot(h, wref[...], preferred_element_type=jnp.float32) + bref[...]
    if relu:
        y = jnp.maximum(y, 0.0)
    if zero0:
        i = pl.program_id(0)
        ri = lax.broadcasted_iota(jnp.int32, y.shape, 0)
        y = jnp.where((i == 0) & (ri == 0), 0.0, y)
        outs[0][...] = y
    else:
        outs[0][...] = y[:, :WB][None]
        outs[1][...] = y[:, WB:][None]


def _tc12(agg, xa, xb, degp, W2, b2, eps, li, relu, zero0):
    body = functools.partial(_tc12_body, relu, zero0, li)
    if zero0:
        out_specs = pl.BlockSpec((RB, 256), lambda i, c: (i, c))
        out_shape = jax.ShapeDtypeStruct((N, D_H), jnp.float32)
    else:
        out_specs = [
            pl.BlockSpec((1, RB, WB), lambda i, c: (c, i, 0)),
            pl.BlockSpec((1, RB, WB), lambda i, c: (c, i, 0)),
        ]
        out_shape = [
            jax.ShapeDtypeStruct((2, N, WB), jnp.float32),
            jax.ShapeDtypeStruct((2, N, WB), jnp.float32),
        ]
    return pl.pallas_call(
        body,
        grid=(GR, 2),
        in_specs=[
            pl.BlockSpec((1, RB, 2 * WB), lambda i, c: (0, i, 0)),
            pl.BlockSpec((1, RB, 2 * WB), lambda i, c: (1, i, 0)),
            pl.BlockSpec((1, RB, WB), lambda i, c: (0, i, 0)),
            pl.BlockSpec((1, RB, WB), lambda i, c: (1, i, 0)),
            pl.BlockSpec((1, RB, WB), lambda i, c: (0, i, 0)),
            pl.BlockSpec((1, RB, WB), lambda i, c: (0, i, 0)),
            pl.BlockSpec((1, RB, WB), lambda i, c: (1, i, 0)),
            pl.BlockSpec((1, RB, WB), lambda i, c: (1, i, 0)),
            pl.BlockSpec((D_H, 256), lambda i, c: (0, c)),
            pl.BlockSpec((1, 256), lambda i, c: (0, c)),
            pl.BlockSpec(memory_space=pltpu.SMEM),
        ],
        out_specs=out_specs,
        out_shape=out_shape,
    )(agg, agg, degp, degp, xa, xb, xa, xb, W2, b2, eps)


def kernel(ndata, edge_index, edge_w, W1, b1, W2, b2, eps):
    x0 = jnp.concatenate([ndata[:, :WB], ndata[:, WB:]], axis=0)
    b1r = b1.reshape(1, D_H)
    b2r = b2.reshape(1, D_H)

    esrc = edge_index[0]
    edst = edge_index[1]
    degp_f, agg0 = _sc_agg0(x0, esrc, edst, edge_w)
    degp = degp_f.reshape(2, N, WB)
    agg0r = agg0.reshape(2, N, WB)
    x0r = x0.reshape(2, N, WB)
    xa1, xb1 = _tc0(agg0r, degp, x0r, W1, b1r, eps)

    agg1 = _sc_agg(xa1.reshape(2 * N, WB), xb1.reshape(2 * N, WB),
                   esrc, edst, edge_w)
    xa2, xb2 = _tc12(agg1.reshape(2, N, 2 * WB), xa1, xb1, degp,
                     W2, b2r, eps, 1, True, False)

    agg2 = _sc_agg(xa2.reshape(2 * N, WB), xb2.reshape(2 * N, WB),
                   esrc, edst, edge_w)
    y = _tc12(agg2.reshape(2, N, 2 * WB), xa2, xb2, degp,
              W2, b2r, eps, 2, False, True)
    return y


# speedup vs baseline: 1.6769x; 1.0184x over previous
import functools

import jax
import jax.numpy as jnp
from jax import lax
from jax.experimental import pallas as pl
from jax.experimental.pallas import tpu as pltpu
from jax.experimental.pallas import tpu_sc as plsc

N = 10000
E = 160000
D_IN = 256
D_H = 512

NC = 2
NS = 16
EPT = E // NS
EPC = E // (NC * NS)
WB = 128
SLAB = 10112
NBUF = 5
NBAT = EPT // 16


def _sc_mesh():
    return plsc.VectorSubcoreMesh(
        core_axis_name="c", subcore_axis_name="s", num_cores=NC, num_subcores=NS
    )


def _zero_zbuf(zbuf):
    zero16 = jnp.zeros((16,), jnp.float32)
    for r in range(16):
        for j in range(WB // 16):
            zbuf[r, pl.ds(j * 16, 16)] = zero16


def _zero_slab(s, zbuf, slab):
    rows = SLAB // NS
    for k in range(rows // 16):
        pltpu.sync_copy(zbuf, slab.at[pl.ds(s * rows + k * 16, 16)])
    pltpu.sync_copy(zbuf.at[pl.ds(0, 8)],
                    slab.at[pl.ds(s * rows + (rows // 16) * 16, 8)])


def _copy_out(c, s, slab, out_hbm, col):
    rows = SLAB // NS
    last = N - (NS - 1) * rows

    @pl.when(s < NS - 1)
    def _():
        pltpu.sync_copy(slab.at[pl.ds(s * rows, rows)],
                        out_hbm.at[pl.ds(c * N + s * rows, rows),
                                   pl.ds(col, WB)])

    @pl.when(s == NS - 1)
    def _():
        pltpu.sync_copy(slab.at[pl.ds((NS - 1) * rows, last)],
                        out_hbm.at[pl.ds(c * N + (NS - 1) * rows, last),
                                   pl.ds(col, WB)])


def _agg_pass(x_hbm, srcb, dstb, wb, gbufs, sbufs, slab, rowbase, sems, ssems):
    def start_gather(bb, k):
        sv = srcb[pl.ds(bb, 16)] + rowbase
        pltpu.async_copy(x_hbm.at[sv], gbufs[k], sems[k])

    def wait_scatter(sl):
        pltpu.make_async_copy(x_hbm.at[pl.ds(0, 16)], sbufs[sl],
                              ssems[sl]).wait()

    for k in range(NBUF):
        start_gather(k * 16, k)

    def round_body(i, carry):
        for k in range(NBUF):
            sl = k % 2
            bb = (i * NBUF + k) * 16
            pltpu.make_async_copy(x_hbm.at[pl.ds(0, 16)], gbufs[k],
                                  sems[k]).wait()
            if k < 2:
                @pl.when(i > 0)
                def _():
                    wait_scatter(sl)
            else:
                wait_scatter(sl)
            dv = dstb[pl.ds(bb, 16)]
            wv = wb[pl.ds(bb, 16)]
            g = gbufs[k]
            sb = sbufs[sl]
            for r in range(16):
                w_r = wv[r]
                for j in range(WB // 16):
                    sb[r, pl.ds(j * 16, 16)] = g[r, pl.ds(j * 16, 16)] * w_r
            pltpu.async_copy(sb, slab.at[dv], ssems[sl], add=True)

            @pl.when(i < NBAT // NBUF - 1)
            def _():
                start_gather(bb + NBUF * 16, k)
        return carry

    lax.fori_loop(0, NBAT // NBUF, round_body, jnp.int32(0))
    wait_scatter(0)
    wait_scatter(1)


def _sc_agg0_body(x_hbm, es_hbm, ed_hbm, ew_hbm, deg_hbm, out_hbm,
                  srcb, dstb, wb, g0, g1, g2, g3, g4, sb0, sb1, zbuf, slab,
                  s0, s1, s2, s3, s4, ss0, ss1):
    c = lax.axis_index("c")
    s = lax.axis_index("s")

    base_d = (c * NS + s) * EPC
    pltpu.sync_copy(ed_hbm.at[pl.ds(base_d, EPC)], dstb.at[pl.ds(0, EPC)])
    dstb[pl.ds(EPC, 16)] = jnp.full((16,), N, jnp.int32)
    onehot = jnp.where(lax.iota(jnp.int32, 16) == 0, 1.0, 0.0).astype(jnp.float32)
    zero16 = jnp.zeros((16,), jnp.float32)
    for r in range(16):
        sb0[r, pl.ds(0, 16)] = onehot
        for j in range(1, WB // 16):
            sb0[r, pl.ds(j * 16, 16)] = zero16
    _zero_zbuf(zbuf)
    _zero_slab(s, zbuf, slab)
    plsc.subcore_barrier()

    def dbody(b, carry):
        dv = dstb[pl.ds(b * 16, 16)]
        pltpu.async_copy(sb0, slab.at[dv], ss0, add=True)
        return carry

    lax.fori_loop(0, (EPC + 15) // 16, dbody, jnp.int32(0))

    def ddrain(b, carry):
        pltpu.make_async_copy(x_hbm.at[pl.ds(0, 16)], sb0, ss0).wait()
        return carry

    lax.fori_loop(0, (EPC + 15) // 16, ddrain, jnp.int32(0))
    plsc.subcore_barrier()
    _copy_out(c, s, slab, deg_hbm, 0)
    plsc.subcore_barrier()

    base_e = s * EPT
    pltpu.sync_copy(es_hbm.at[pl.ds(base_e, EPT)], srcb)
    pltpu.sync_copy(ed_hbm.at[pl.ds(base_e, EPT)], dstb)
    pltpu.sync_copy(ew_hbm.at[pl.ds(base_e, EPT)], wb)

    _zero_slab(s, zbuf, slab)
    plsc.subcore_barrier()

    _agg_pass(x_hbm, srcb, dstb, wb, [g0, g1, g2, g3, g4], [sb0, sb1], slab,
              c * N, [s0, s1, s2, s3, s4], [ss0, ss1])
    plsc.subcore_barrier()
    _copy_out(c, s, slab, out_hbm, 0)


def _sc_agg0(x2, esrc, edst, edge_w):
    f = pl.kernel(
        _sc_agg0_body,
        out_type=[jax.ShapeDtypeStruct((2 * N, WB), jnp.float32),
                  jax.ShapeDtypeStruct((2 * N, WB), jnp.float32)],
        mesh=_sc_mesh(),
        scratch_types=(
            [pltpu.VMEM((EPT,), jnp.int32),
             pltpu.VMEM((EPT,), jnp.int32),
             pltpu.VMEM((EPT,), jnp.float32)]
            + [pltpu.VMEM((16, WB), jnp.float32)] * (NBUF + 3)
            + [pltpu.VMEM_SHARED((SLAB, WB), jnp.float32)]
            + [pltpu.SemaphoreType.DMA] * (NBUF + 2)
        ),
    )
    return f(x2, esrc, edst, edge_w)


def _sc_agg_body(xa_hbm, xb_hbm, es_hbm, ed_hbm, ew_hbm, out_hbm,
                 srcb, dstb, wb, g0, g1, g2, g3, g4, sb0, sb1, zbuf, slab,
                 s0, s1, s2, s3, s4, ss0, ss1):
    c = lax.axis_index("c")
    s = lax.axis_index("s")
    base_e = s * EPT
    pltpu.sync_copy(es_hbm.at[pl.ds(base_e, EPT)], srcb)
    pltpu.sync_copy(ed_hbm.at[pl.ds(base_e, EPT)], dstb)
    pltpu.sync_copy(ew_hbm.at[pl.ds(base_e, EPT)], wb)
    _zero_zbuf(zbuf)

    for p, x_hbm in enumerate((xa_hbm, xb_hbm)):
        _zero_slab(s, zbuf, slab)
        plsc.subcore_barrier()
        _agg_pass(x_hbm, srcb, dstb, wb, [g0, g1, g2, g3, g4], [sb0, sb1],
                  slab, c * N, [s0, s1, s2, s3, s4], [ss0, ss1])
        plsc.subcore_barrier()
        _copy_out(c, s, slab, out_hbm, p * WB)
        plsc.subcore_barrier()


def _sc_agg(xa, xb, esrc, edst, edge_w):
    f = pl.kernel(
        _sc_agg_body,
        out_type=jax.ShapeDtypeStruct((2 * N, 2 * WB), jnp.float32),
        mesh=_sc_mesh(),
        scratch_types=(
            [pltpu.VMEM((EPT,), jnp.int32),
             pltpu.VMEM((EPT,), jnp.int32),
             pltpu.VMEM((EPT,), jnp.float32)]
            + [pltpu.VMEM((16, WB), jnp.float32)] * (NBUF + 3)
            + [pltpu.VMEM_SHARED((SLAB, WB), jnp.float32)]
            + [pltpu.SemaphoreType.DMA] * (NBUF + 2)
        ),
    )
    return f(xa, xb, esrc, edst, edge_w)


RB = 1000
GR = N // RB


def _tc0_body(aggA, aggB, degA, degB, xA, xB, wref, bref, eref, outa, outb):
    deg = degA[0][:, :1] + degB[0][:, :1]
    rdeg = 1.0 / jnp.maximum(deg, 1.0)
    mean = jnp.concatenate([aggA[0], aggB[0]], axis=1) * rdeg
    x = jnp.concatenate([xA[0], xB[0]], axis=1)
    h = (1.0 + eref[0]) * x + mean
    y = jnp.dot(h, wref[...], preferred_element_type=jnp.float32) + bref[...]
    y = jnp.maximum(y, 0.0)
    outa[...] = y[:, :WB][None]
    outb[...] = y[:, WB:][None]


def _tc0(agg0, degp, x0, W1, b1, eps):
    return pl.pallas_call(
        _tc0_body,
        grid=(GR, 2),
        in_specs=[
            pl.BlockSpec((1, RB, WB), lambda i, c: (0, i, 0)),
            pl.BlockSpec((1, RB, WB), lambda i, c: (1, i, 0)),
            pl.BlockSpec((1, RB, WB), lambda i, c: (0, i, 0)),
            pl.BlockSpec((1, RB, WB), lambda i, c: (1, i, 0)),
            pl.BlockSpec((1, RB, WB), lambda i, c: (0, i, 0)),
            pl.BlockSpec((1, RB, WB), lambda i, c: (1, i, 0)),
            pl.BlockSpec((D_IN, 256), lambda i, c: (0, c)),
            pl.BlockSpec((1, 256), lambda i, c: (0, c)),
            pl.BlockSpec(memory_space=pltpu.SMEM),
        ],
        out_specs=[
            pl.BlockSpec((1, RB, WB), lambda i, c: (c, i, 0)),
            pl.BlockSpec((1, RB, WB), lambda i, c: (c, i, 0)),
        ],
        out_shape=[
            jax.ShapeDtypeStruct((2, N, WB), jnp.float32),
            jax.ShapeDtypeStruct((2, N, WB), jnp.float32),
        ],
    )(agg0, agg0, degp, degp, x0, x0, W1, b1, eps)


def _tc12_body(relu, zero0, li,
               aggA, aggB, degA, degB, xa0, xb0, xa1, xb1,
               wref, bref, eref, *outs):
    deg = degA[0][:, :1] + degB[0][:, :1]
    rdeg = 1.0 / jnp.maximum(deg, 1.0)
    mean = jnp.concatenate([aggA[0], aggB[0]], axis=1) * rdeg
    x = jnp.concatenate([xa0[0], xb0[0], xa1[0], xb1[0]], axis=1)
    h = (1.0 + eref[li]) * x + mean
    y = jnp.dot(h, wref[...], preferred_element_type=jnp.float32) + bref[...]
    if relu:
        y = jnp.maximum(y, 0.0)
    if zero0:
        i = pl.program_id(0)
        ri = lax.broadcasted_iota(jnp.int32, y.shape, 0)
        y = jnp.where((i == 0) & (ri == 0), 0.0, y)
        outs[0][...] = y
    else:
        outs[0][...] = y[:, :WB][None]
        outs[1][...] = y[:, WB:][None]


def _tc12(agg, xa, xb, degp, W2, b2, eps, li, relu, zero0):
    body = functools.partial(_tc12_body, relu, zero0, li)
    if zero0:
        out_specs = pl.BlockSpec((RB, 256), lambda i, c: (i, c))
        out_shape = jax.ShapeDtypeStruct((N, D_H), jnp.float32)
    else:
        out_specs = [
            pl.BlockSpec((1, RB, WB), lambda i, c: (c, i, 0)),
            pl.BlockSpec((1, RB, WB), lambda i, c: (c, i, 0)),
        ]
        out_shape = [
            jax.ShapeDtypeStruct((2, N, WB), jnp.float32),
            jax.ShapeDtypeStruct((2, N, WB), jnp.float32),
        ]
    return pl.pallas_call(
        body,
        grid=(GR, 2),
        in_specs=[
            pl.BlockSpec((1, RB, 2 * WB), lambda i, c: (0, i, 0)),
            pl.BlockSpec((1, RB, 2 * WB), lambda i, c: (1, i, 0)),
            pl.BlockSpec((1, RB, WB), lambda i, c: (0, i, 0)),
            pl.BlockSpec((1, RB, WB), lambda i, c: (1, i, 0)),
            pl.BlockSpec((1, RB, WB), lambda i, c: (0, i, 0)),
            pl.BlockSpec((1, RB, WB), lambda i, c: (0, i, 0)),
            pl.BlockSpec((1, RB, WB), lambda i, c: (1, i, 0)),
            pl.BlockSpec((1, RB, WB), lambda i, c: (1, i, 0)),
            pl.BlockSpec((D_H, 256), lambda i, c: (0, c)),
            pl.BlockSpec((1, 256), lambda i, c: (0, c)),
            pl.BlockSpec(memory_space=pltpu.SMEM),
        ],
        out_specs=out_specs,
        out_shape=out_shape,
    )(agg, agg, degp, degp, xa, xb, xa, xb, W2, b2, eps)


def kernel(ndata, edge_index, edge_w, W1, b1, W2, b2, eps):
    x0 = jnp.concatenate([ndata[:, :WB], ndata[:, WB:]], axis=0)
    b1r = b1.reshape(1, D_H)
    b2r = b2.reshape(1, D_H)

    esrc = edge_index[0]
    edst = edge_index[1]
    degp_f, agg0 = _sc_agg0(x0, esrc, edst, edge_w)
    degp = degp_f.reshape(2, N, WB)
    agg0r = agg0.reshape(2, N, WB)
    x0r = x0.reshape(2, N, WB)
    xa1, xb1 = _tc0(agg0r, degp, x0r, W1, b1r, eps)

    agg1 = _sc_agg(xa1.reshape(2 * N, WB), xb1.reshape(2 * N, WB),
                   esrc, edst, edge_w)
    xa2, xb2 = _tc12(agg1.reshape(2, N, 2 * WB), xa1, xb1, degp,
                     W2, b2r, eps, 1, True, False)

    agg2 = _sc_agg(xa2.reshape(2 * N, WB), xb2.reshape(2 * N, WB),
                   esrc, edst, edge_w)
    y = _tc12(agg2.reshape(2, N, 2 * WB), xa2, xb2, degp,
              W2, b2r, eps, 2, False, True)
    return y
